# baseline (speedup 1.0000x reference)
import functools

import jax
import jax.numpy as jnp
from jax import lax
from jax.experimental import pallas as pl
from jax.experimental.pallas import tpu as pltpu
from jax.experimental.pallas import tpu_sc as plsc

BLK = 4000
CHUNK = 64


def kernel(color_idx, table):
    (B,) = color_idx.shape
    V, D = table.shape
    info = plsc.get_sparse_core_info()
    NC, NS = info.num_cores, info.num_subcores
    NW = NC * NS
    L = info.num_lanes
    b_per_w = B // NW
    nch = b_per_w // CHUNK

    idx1 = color_idx.astype(jnp.int32)

    V2 = V // 2
    WR = 1000
    nwh = V2 // WR

    mesh = plsc.VectorSubcoreMesh(core_axis_name="c", subcore_axis_name="s")

    V4 = V // 4
    compact_a = table[:V2].reshape(V4, 2 * D)
    compact_b = table[V2:].reshape(V4, 2 * D)

    @functools.partial(
        pl.kernel,
        mesh=mesh,
        out_type=jax.ShapeDtypeStruct((B, D), jnp.float32),
        scratch_types=[
            pltpu.VMEM((b_per_w,), jnp.int32),
            pltpu.VMEM((nch, CHUNK), jnp.int32),
            pltpu.VMEM((2 * CHUNK, 2 * D), jnp.float32),
            pltpu.VMEM((2 * CHUNK, 2 * D), jnp.float32),
            pltpu.VMEM((b_per_w, D), jnp.float32),
            pltpu.SemaphoreType.DMA,
            pltpu.SemaphoreType.DMA,
        ],
        compiler_params=pltpu.CompilerParams(
            use_tc_tiling_on_sc=True, needs_layout_passes=False
        ),
    )
    def gather(
        idx_hbm, ca_hbm, cb_hbm, out_hbm,
        idx_v, pidx_v, pairs_a, pairs_b, rows_v, sem_a, sem_b,
    ):
        wid = lax.axis_index("s") * NC + lax.axis_index("c")
        base = wid * b_per_w
        pltpu.sync_copy(idx_hbm.at[pl.ds(base, b_per_w)], idx_v)
        for j in range(nch):
            for g in range(CHUNK // L):
                iv = idx_v[pl.ds(j * CHUNK + g * L, L)]
                hi = jnp.where(iv >= V2, jnp.int32(V2), jnp.int32(0))
                pidx_v[j, pl.ds(g * L, L)] = lax.shift_right_logical(iv - hi, 1)
        bufs = (pairs_a, pairs_b)
        sems = (sem_a, sem_b)

        def issue(j):
            pltpu.async_copy(
                ca_hbm.at[pidx_v.at[j]],
                bufs[j % 2].at[pl.ds(0, CHUNK)],
                sems[j % 2],
            )
            pltpu.async_copy(
                cb_hbm.at[pidx_v.at[j]],
                bufs[j % 2].at[pl.ds(CHUNK, CHUNK)],
                sems[j % 2],
            )

        def wait(j):
            for _ in range(2):
                pltpu.make_async_copy(
                    ca_hbm.at[pidx_v.at[j]],
                    bufs[j % 2].at[pl.ds(0, CHUNK)],
                    sems[j % 2],
                ).wait()

        issue(0)
        for j in range(nch):
            if j + 1 < nch:
                issue(j + 1)
            wait(j)
            pv = bufs[j % 2]
            for g in range(CHUNK // L):
                iv = idx_v[pl.ds(j * CHUNK + g * L, L)]
                rowi = lax.iota(jnp.int32, L) + g * L
                srow = rowi + jnp.where(
                    iv >= V2, jnp.int32(CHUNK), jnp.int32(0)
                )
                orow = rowi + j * CHUNK
                colb = lax.mul(lax.bitwise_and(iv, 1), D)
                zero = jnp.zeros((L,), jnp.int32)

                def body(e, carry, pv=pv, srow=srow, orow=orow, colb=colb, zero=zero):
                    v = plsc.load_gather(pv, [srow, colb + e])
                    plsc.store_scatter(rows_v, [orow, zero + e], v)
                    return carry

                lax.fori_loop(0, D, body, 0)
        pltpu.sync_copy(rows_v, out_hbm.at[pl.ds(base, b_per_w)])

    return gather(idx1, compact_a, compact_b)


# speedup vs baseline: 2.5893x; 2.5893x over previous
import functools

import jax
import jax.numpy as jnp
from jax import lax
from jax.experimental import pallas as pl
from jax.experimental.pallas import tpu as pltpu
from jax.experimental.pallas import tpu_sc as plsc


def kernel(color_idx, table):
    (B,) = color_idx.shape
    V, D = table.shape
    info = plsc.get_sparse_core_info()
    NC, NS = info.num_cores, info.num_subcores
    NW = NC * NS
    L = info.num_lanes
    b_per_w = B // NW

    idx1 = color_idx.astype(jnp.int32)

    mesh = plsc.VectorSubcoreMesh(core_axis_name="c", subcore_axis_name="s")

    @functools.partial(
        pl.kernel,
        mesh=mesh,
        out_type=jax.ShapeDtypeStruct((B, D), jnp.float32),
        scratch_types=[
            pltpu.VMEM((b_per_w,), jnp.int32),
            pltpu.VMEM((b_per_w, D), jnp.float32),
            pltpu.SemaphoreType.DMA,
        ],
        compiler_params=pltpu.CompilerParams(use_tc_tiling_on_sc=True),
    )
    def emb(idx_hbm, table_hbm, out_hbm, idx_v, rows_v, sem):
        wid = lax.axis_index("s") * NC + lax.axis_index("c")
        base = wid * b_per_w
        pltpu.sync_copy(idx_hbm.at[pl.ds(base, b_per_w)], idx_v)

        def group(g, carry):
            iv = idx_v[pl.ds(g * L, L)]
            for l in range(L):
                i = iv[l]
                pltpu.async_copy(table_hbm.at[i], rows_v.at[g * L + l], sem)
            return carry

        lax.fori_loop(0, b_per_w // L, group, 0)
        pltpu.make_async_copy(
            table_hbm.at[pl.ds(0, b_per_w)], rows_v, sem
        ).wait()
        pltpu.sync_copy(rows_v, out_hbm.at[pl.ds(base, b_per_w)])

    return emb(idx1, table)
